# baseline (speedup 1.0000x reference)
import functools

import jax
import jax.numpy as jnp
from jax import lax
from jax.experimental import pallas as pl
from jax.experimental.pallas import tpu as pltpu
from jax.experimental.pallas import tpu_sc as plsc

N = 10000
E = 320000
C = 128
ED = 16
DD = 128
DO = 2
SLOPE = 0.2
NL = 0.01

NC = 2
NS = 16
L = 16

R = C
NP = 10240
EPT = E // (NC * NS)
B = 40
RPT = NP // NS


def _node_xfm_body(x_ref, wl_ref, bl_ref, wr_ref, br_ref, xl_ref, xr_ref):
    xb = x_ref[...]
    xl_ref[...] = jnp.dot(xb, wl_ref[...], preferred_element_type=jnp.float32) + bl_ref[...]
    xr_ref[...] = jnp.dot(xb, wr_ref[...], preferred_element_type=jnp.float32) + br_ref[...]


def _node_xfm(x, Wl, bl, Wr, br):
    bn = 1000
    grid = (N // bn,)
    return pl.pallas_call(
        _node_xfm_body,
        grid=grid,
        in_specs=[
            pl.BlockSpec((bn, C), lambda i: (i, 0)),
            pl.BlockSpec((C, C), lambda i: (0, 0)),
            pl.BlockSpec((1, C), lambda i: (0, 0)),
            pl.BlockSpec((C, C), lambda i: (0, 0)),
            pl.BlockSpec((1, C), lambda i: (0, 0)),
        ],
        out_specs=[
            pl.BlockSpec((bn, C), lambda i: (i, 0)),
            pl.BlockSpec((bn, C), lambda i: (i, 0)),
        ],
        out_shape=[
            jax.ShapeDtypeStruct((N, C), jnp.float32),
            jax.ShapeDtypeStruct((N, C), jnp.float32),
        ],
    )(x, Wl, bl.reshape(1, C), Wr, br.reshape(1, C))


def _eaw_body(ea_ref, we_ref, out_ref):
    out_ref[...] = jnp.dot(ea_ref[...], we_ref[...], preferred_element_type=jnp.float32)


def _eaw(edge_attr, We):
    be = 4000
    grid = (E // be,)
    return pl.pallas_call(
        _eaw_body,
        grid=grid,
        in_specs=[
            pl.BlockSpec((be, ED), lambda i: (i, 0)),
            pl.BlockSpec((ED, C), lambda i: (0, 0)),
        ],
        out_specs=pl.BlockSpec((be, C), lambda i: (i, 0)),
        out_shape=jax.ShapeDtypeStruct((E, C), jnp.float32),
    )(edge_attr, We)


def _norm_h(acc_ref, s0_ref, bo_ref):
    acc_blk = acc_ref[...]
    num = acc_blk[0] + acc_blk[1]
    s = jnp.sum(s0_ref[...], axis=(0, 1))
    h = num / (s + 1e-16) + bo_ref[...]
    return jnp.where(h > 0, h, NL * h)


def _head_body(acc_ref, s0_ref, bo_ref, wl_ref, bl_ref, wr_ref, br_ref, xl_ref, xr_ref):
    h = _norm_h(acc_ref, s0_ref, bo_ref)
    xl_ref[...] = jnp.dot(h, wl_ref[...], preferred_element_type=jnp.float32) + bl_ref[...]
    xr_ref[...] = jnp.dot(h, wr_ref[...], preferred_element_type=jnp.float32) + br_ref[...]


def _head(acc, s0, bo, Wl, bl, Wr, br):
    bn = 1024
    grid = (NP // bn,)
    return pl.pallas_call(
        _head_body,
        grid=grid,
        in_specs=[
            pl.BlockSpec((NC, bn, C), lambda i: (0, i, 0)),
            pl.BlockSpec((NC, NS, bn, 1), lambda i: (0, 0, i, 0)),
            pl.BlockSpec((1, C), lambda i: (0, 0)),
            pl.BlockSpec((C, C), lambda i: (0, 0)),
            pl.BlockSpec((1, C), lambda i: (0, 0)),
            pl.BlockSpec((C, C), lambda i: (0, 0)),
            pl.BlockSpec((1, C), lambda i: (0, 0)),
        ],
        out_specs=[
            pl.BlockSpec((bn, C), lambda i: (i, 0)),
            pl.BlockSpec((bn, C), lambda i: (i, 0)),
        ],
        out_shape=[
            jax.ShapeDtypeStruct((NP, C), jnp.float32),
            jax.ShapeDtypeStruct((NP, C), jnp.float32),
        ],
    )(acc, s0.reshape(NC, NS, NP, 1), bo.reshape(1, C), Wl, bl.reshape(1, C),
      Wr, br.reshape(1, C))


def _final_body(acc_ref, s0_ref, bo_ref, wd1_ref, bd1_ref, wd2_ref, bd2_ref,
                out_ref):
    h = _norm_h(acc_ref, s0_ref, bo_ref)
    d = jnp.dot(h, wd1_ref[...], preferred_element_type=jnp.float32) + bd1_ref[...]
    d = jnp.where(d > 0, d, NL * d)
    out_ref[...] = jnp.dot(d, wd2_ref[...], preferred_element_type=jnp.float32) + bd2_ref[...]


def _final(acc, s0, bo, Wd1, bd1, Wd2, bd2):
    bn = 1024
    grid = (NP // bn,)
    return pl.pallas_call(
        _final_body,
        grid=grid,
        in_specs=[
            pl.BlockSpec((NC, bn, C), lambda i: (0, i, 0)),
            pl.BlockSpec((NC, NS, bn, 1), lambda i: (0, 0, i, 0)),
            pl.BlockSpec((1, C), lambda i: (0, 0)),
            pl.BlockSpec((C, DD), lambda i: (0, 0)),
            pl.BlockSpec((1, DD), lambda i: (0, 0)),
            pl.BlockSpec((DD, DO), lambda i: (0, 0)),
            pl.BlockSpec((1, DO), lambda i: (0, 0)),
        ],
        out_specs=pl.BlockSpec((bn, DO), lambda i: (i, 0)),
        out_shape=jax.ShapeDtypeStruct((NP, DO), jnp.float32),
    )(acc, s0.reshape(NC, NS, NP, 1), bo.reshape(1, C), Wd1,
      bd1.reshape(1, DD), Wd2, bd2.reshape(1, DO))


NCHUNK = EPT // B


def _edge_pass_body(src_hbm, dst_hbm, xl_hbm, xr_hbm, ea_hbm, att_hbm,
                    out_hbm, s0_hbm,
                    src_v0, dst_v0, src_v1, dst_v1,
                    xj_v0, ea_v0, xj_v1, ea_v1,
                    row_v, w_v, att_v, s0_v, acc_sh,
                    s_si0, s_di0, s_gj0, s_gi0, s_ge0,
                    s_si1, s_di1, s_gj1, s_gi1, s_ge1):
    cid = lax.axis_index("c")
    sid = lax.axis_index("s")

    bufs = (
        (src_v0, dst_v0, xj_v0, ea_v0, s_si0, s_di0, s_gj0, s_gi0, s_ge0),
        (src_v1, dst_v1, xj_v1, ea_v1, s_si1, s_di1, s_gj1, s_gi1, s_ge1),
    )

    def _zrow(i, carry):
        for cc in range(R // L):
            row_v[i, cc * L:(cc + 1) * L] = jnp.zeros((L,), jnp.float32)
        return carry
    lax.fori_loop(0, B, _zrow, 0)
    row0 = sid * RPT
    def _zcp(j, carry):
        pltpu.sync_copy(row_v, acc_sh.at[pl.ds(row0 + j * B, B)])
        return carry
    lax.fori_loop(0, RPT // B, _zcp, 0)

    def _zs(i, carry):
        s0_v[pl.ds(i * L, L)] = jnp.zeros((L,), jnp.float32)
        return carry
    lax.fori_loop(0, NP // L, _zs, 0)
    plsc.subcore_barrier()

    pltpu.sync_copy(att_hbm, att_v)
    att_c = [att_v[cc * L:(cc + 1) * L] for cc in range(C // L)]
    lane = lax.iota(jnp.int32, L)
    lane0 = lane == 0
    bfly = [jnp.bitwise_xor(lane, sh).reshape(L, 1) for sh in (8, 4, 2, 1)]
    gd = lax.GatherDimensionNumbers(
        offset_dims=(), collapsed_slice_dims=(0,), start_index_map=(0,))

    def _xl_sum(v):
        for idx in bfly:
            v = v + lax.gather(v, idx, gd, slice_sizes=(1,),
                               mode=lax.GatherScatterMode.PROMISE_IN_BOUNDS)
        return v

    ebase = (cid * NS + sid) * EPT

    def _issue_idx(k, b):
        eoff = pl.multiple_of(ebase + k * B, 8)
        pltpu.make_async_copy(src_hbm.at[pl.ds(eoff, B)], b[0], b[4]).start()
        pltpu.make_async_copy(dst_hbm.at[pl.ds(eoff, B)], b[1], b[5]).start()
        pltpu.make_async_copy(ea_hbm.at[pl.ds(eoff, B)], b[3], b[8]).start()

    def _wait_idx(b):
        pltpu.make_async_copy(src_hbm.at[pl.ds(0, B)], b[0], b[4]).wait()
        pltpu.make_async_copy(dst_hbm.at[pl.ds(0, B)], b[1], b[5]).wait()
        pltpu.make_async_copy(ea_hbm.at[pl.ds(0, B)], b[3], b[8]).wait()

    def _issue_data(k, b):
        pltpu.make_async_copy(xl_hbm.at[b[0]], b[2], b[6]).start()
        pltpu.make_async_copy(xr_hbm.at[b[1]], b[3], b[7]).start(add=True)

    def _wait_data(b):
        pltpu.make_async_copy(xl_hbm.at[b[0]], b[2], b[6]).wait()
        pltpu.make_async_copy(xr_hbm.at[b[1]], b[3], b[7]).wait()

    def _compute_scatter(b):
        dst_b, xj_b, ea_b = b[1], b[2], b[3]

        def _edge(e, ecarry):
            acc = jnp.zeros((L,), jnp.float32)
            xjs = []
            for cc in range(C // L):
                vj = xj_b[e, cc * L:(cc + 1) * L]
                ve = ea_b[e, cc * L:(cc + 1) * L]
                v = vj + ve
                v = jnp.where(v > 0, v, SLOPE * v)
                acc = acc + v * att_c[cc]
                xjs.append(vj)
            w = jnp.exp(_xl_sum(acc))
            for cc in range(C // L):
                row_v[e, cc * L:(cc + 1) * L] = xjs[cc] * w
            plsc.store_scatter(w_v, [jnp.full((L,), e, jnp.int32)], w,
                               mask=lane0)
            return ecarry
        lax.fori_loop(0, B, _edge, 0, unroll=2)

        for g in range((B + L - 1) // L):
            off = min(g * L, B - L)
            wv = w_v[off:off + L]
            dv = dst_b[off:off + L]
            if off == g * L:
                plsc.addupdate_scatter(s0_v, [dv], wv)
            else:
                plsc.addupdate_scatter(s0_v, [dv], wv,
                                       mask=lane >= (g * L - off))

        pltpu.sync_copy(row_v, acc_sh.at[dst_b], add=True)

    _issue_idx(0, bufs[0])
    _issue_idx(1, bufs[1])
    _wait_idx(bufs[0])
    _issue_data(0, bufs[0])

    def _body(k2, carry):
        for slot in (0, 1):
            k = k2 * 2 + slot
            b = bufs[slot]
            o = bufs[1 - slot]

            @pl.when(k < NCHUNK - 1)
            def _():
                _wait_idx(o)
                _issue_data(k + 1, o)

            _wait_data(b)
            _compute_scatter(b)

            @pl.when(k < NCHUNK - 2)
            def _():
                _issue_idx(k + 2, b)
        return carry
    lax.fori_loop(0, NCHUNK // 2, _body, 0)

    pltpu.sync_copy(s0_v, s0_hbm.at[cid, sid])

    plsc.subcore_barrier()
    def _wb(j, carry):
        r0 = row0 + j * B
        pltpu.sync_copy(acc_sh.at[pl.ds(r0, B)],
                        out_hbm.at[cid, pl.ds(r0, B)])
        return carry
    lax.fori_loop(0, RPT // B, _wb, 0)


@functools.partial(
    pl.kernel,
    mesh=plsc.VectorSubcoreMesh(core_axis_name="c", subcore_axis_name="s"),
    compiler_params=pltpu.CompilerParams(needs_layout_passes=False),
    out_type=[
        jax.ShapeDtypeStruct((NC, NP, R), jnp.float32),
        jax.ShapeDtypeStruct((NC, NS, NP), jnp.float32),
    ],
    scratch_types=[
        pltpu.VMEM((B,), jnp.int32),
        pltpu.VMEM((B,), jnp.int32),
        pltpu.VMEM((B,), jnp.int32),
        pltpu.VMEM((B,), jnp.int32),
        pltpu.VMEM((B, C), jnp.float32),
        pltpu.VMEM((B, C), jnp.float32),
        pltpu.VMEM((B, C), jnp.float32),
        pltpu.VMEM((B, C), jnp.float32),
        pltpu.VMEM((B, C), jnp.float32),
        pltpu.VMEM((B,), jnp.float32),
        pltpu.VMEM((C,), jnp.float32),
        pltpu.VMEM((NP,), jnp.float32),
        pltpu.VMEM_SHARED((NP, R), jnp.float32),
    ] + [pltpu.SemaphoreType.DMA] * 10,
)
def _edge_pass(src_hbm, dst_hbm, xl_hbm, xr_hbm, ea_hbm, att_hbm, out_hbm,
               s0_hbm,
               src_v0, dst_v0, src_v1, dst_v1,
               xj_v0, ea_v0, xj_v1, ea_v1,
               row_v, w_v, att_v, s0_v, acc_sh,
               s_si0, s_di0, s_gj0, s_gi0, s_ge0,
               s_si1, s_di1, s_gj1, s_gi1, s_ge1):
    _edge_pass_body(src_hbm, dst_hbm, xl_hbm, xr_hbm, ea_hbm, att_hbm,
                    out_hbm, s0_hbm,
                    src_v0, dst_v0, src_v1, dst_v1,
                    xj_v0, ea_v0, xj_v1, ea_v1,
                    row_v, w_v, att_v, s0_v, acc_sh,
                    s_si0, s_di0, s_gj0, s_gi0, s_ge0,
                    s_si1, s_di1, s_gj1, s_gi1, s_ge1)


def kernel(x, edge_index, edge_attr, Wl1, bl1, Wr1, br1, We1, att1, bo1,
           Wl2, bl2, Wr2, br2, We2, att2, bo2, Wd1, bd1, Wd2, bd2):
    src = edge_index[0]
    dst = edge_index[1]

    xl1, xr1 = _node_xfm(x, Wl1, bl1, Wr1, br1)
    eaw1 = _eaw(edge_attr, We1)
    acc1, s01 = _edge_pass(src, dst, xl1, xr1, eaw1, att1.reshape(C))

    xl2, xr2 = _head(acc1, s01, bo1, Wl2, bl2, Wr2, br2)
    eaw2 = _eaw(edge_attr, We2)
    acc2, s02 = _edge_pass(src, dst, xl2, xr2, eaw2, att2.reshape(C))

    return _final(acc2, s02, bo2, Wd1, bd1, Wd2, bd2)[:N]


# speedup vs baseline: 1.0548x; 1.0548x over previous
import functools

import jax
import jax.numpy as jnp
from jax import lax
from jax.experimental import pallas as pl
from jax.experimental.pallas import tpu as pltpu
from jax.experimental.pallas import tpu_sc as plsc

N = 10000
E = 320000
C = 128
ED = 16
DD = 128
DO = 2
SLOPE = 0.2
NL = 0.01

NC = 2
NS = 16
L = 16

R = C
NP = 10240
EPT = E // (NC * NS)
B = 40
RPT = NP // NS


def _node_xfm_body(x_ref, wl_ref, bl_ref, wr_ref, br_ref, xl_ref, xr_ref):
    xb = x_ref[...]
    xl_ref[...] = jnp.dot(xb, wl_ref[...], preferred_element_type=jnp.float32) + bl_ref[...]
    xr_ref[...] = jnp.dot(xb, wr_ref[...], preferred_element_type=jnp.float32) + br_ref[...]


def _node_xfm(x, Wl, bl, Wr, br):
    bn = 1000
    grid = (N // bn,)
    return pl.pallas_call(
        _node_xfm_body,
        grid=grid,
        in_specs=[
            pl.BlockSpec((bn, C), lambda i: (i, 0)),
            pl.BlockSpec((C, C), lambda i: (0, 0)),
            pl.BlockSpec((1, C), lambda i: (0, 0)),
            pl.BlockSpec((C, C), lambda i: (0, 0)),
            pl.BlockSpec((1, C), lambda i: (0, 0)),
        ],
        out_specs=[
            pl.BlockSpec((bn, C), lambda i: (i, 0)),
            pl.BlockSpec((bn, C), lambda i: (i, 0)),
        ],
        out_shape=[
            jax.ShapeDtypeStruct((N, C), jnp.float32),
            jax.ShapeDtypeStruct((N, C), jnp.float32),
        ],
    )(x, Wl, bl.reshape(1, C), Wr, br.reshape(1, C))


def _eaw_body(ea_ref, we1_ref, we2_ref, out1_ref, out2_ref):
    eb = ea_ref[...]
    out1_ref[...] = jnp.dot(eb, we1_ref[...], preferred_element_type=jnp.float32)
    out2_ref[...] = jnp.dot(eb, we2_ref[...], preferred_element_type=jnp.float32)


def _eaw(edge_attr, We1, We2):
    be = 4000
    grid = (E // be,)
    return pl.pallas_call(
        _eaw_body,
        grid=grid,
        in_specs=[
            pl.BlockSpec((be, ED), lambda i: (i, 0)),
            pl.BlockSpec((ED, C), lambda i: (0, 0)),
            pl.BlockSpec((ED, C), lambda i: (0, 0)),
        ],
        out_specs=[
            pl.BlockSpec((be, C), lambda i: (i, 0)),
            pl.BlockSpec((be, C), lambda i: (i, 0)),
        ],
        out_shape=[
            jax.ShapeDtypeStruct((E, C), jnp.float32),
            jax.ShapeDtypeStruct((E, C), jnp.float32),
        ],
    )(edge_attr, We1, We2)


def _norm_h(acc_ref, s0_ref, bo_ref):
    acc_blk = acc_ref[...]
    num = acc_blk[0] + acc_blk[1]
    s = jnp.sum(s0_ref[...], axis=(0, 1))
    h = num / (s + 1e-16) + bo_ref[...]
    return jnp.where(h > 0, h, NL * h)


def _head_body(acc_ref, s0_ref, bo_ref, wl_ref, bl_ref, wr_ref, br_ref, xl_ref, xr_ref):
    h = _norm_h(acc_ref, s0_ref, bo_ref)
    xl_ref[...] = jnp.dot(h, wl_ref[...], preferred_element_type=jnp.float32) + bl_ref[...]
    xr_ref[...] = jnp.dot(h, wr_ref[...], preferred_element_type=jnp.float32) + br_ref[...]


def _head(acc, s0, bo, Wl, bl, Wr, br):
    bn = 1024
    grid = (NP // bn,)
    return pl.pallas_call(
        _head_body,
        grid=grid,
        in_specs=[
            pl.BlockSpec((NC, bn, C), lambda i: (0, i, 0)),
            pl.BlockSpec((NC, NS, bn, 1), lambda i: (0, 0, i, 0)),
            pl.BlockSpec((1, C), lambda i: (0, 0)),
            pl.BlockSpec((C, C), lambda i: (0, 0)),
            pl.BlockSpec((1, C), lambda i: (0, 0)),
            pl.BlockSpec((C, C), lambda i: (0, 0)),
            pl.BlockSpec((1, C), lambda i: (0, 0)),
        ],
        out_specs=[
            pl.BlockSpec((bn, C), lambda i: (i, 0)),
            pl.BlockSpec((bn, C), lambda i: (i, 0)),
        ],
        out_shape=[
            jax.ShapeDtypeStruct((NP, C), jnp.float32),
            jax.ShapeDtypeStruct((NP, C), jnp.float32),
        ],
    )(acc, s0.reshape(NC, NS, NP, 1), bo.reshape(1, C), Wl, bl.reshape(1, C),
      Wr, br.reshape(1, C))


def _final_body(acc_ref, s0_ref, bo_ref, wd1_ref, bd1_ref, wd2_ref, bd2_ref,
                out_ref):
    h = _norm_h(acc_ref, s0_ref, bo_ref)
    d = jnp.dot(h, wd1_ref[...], preferred_element_type=jnp.float32) + bd1_ref[...]
    d = jnp.where(d > 0, d, NL * d)
    out_ref[...] = jnp.dot(d, wd2_ref[...], preferred_element_type=jnp.float32) + bd2_ref[...]


def _final(acc, s0, bo, Wd1, bd1, Wd2, bd2):
    bn = 1024
    grid = (NP // bn,)
    return pl.pallas_call(
        _final_body,
        grid=grid,
        in_specs=[
            pl.BlockSpec((NC, bn, C), lambda i: (0, i, 0)),
            pl.BlockSpec((NC, NS, bn, 1), lambda i: (0, 0, i, 0)),
            pl.BlockSpec((1, C), lambda i: (0, 0)),
            pl.BlockSpec((C, DD), lambda i: (0, 0)),
            pl.BlockSpec((1, DD), lambda i: (0, 0)),
            pl.BlockSpec((DD, DO), lambda i: (0, 0)),
            pl.BlockSpec((1, DO), lambda i: (0, 0)),
        ],
        out_specs=pl.BlockSpec((bn, DO), lambda i: (i, 0)),
        out_shape=jax.ShapeDtypeStruct((NP, DO), jnp.float32),
    )(acc, s0.reshape(NC, NS, NP, 1), bo.reshape(1, C), Wd1,
      bd1.reshape(1, DD), Wd2, bd2.reshape(1, DO))


NCHUNK = EPT // B


def _edge_pass_body(src_hbm, dst_hbm, xl_hbm, xr_hbm, ea_hbm, att_hbm,
                    out_hbm, s0_hbm,
                    src_v0, dst_v0, src_v1, dst_v1,
                    xj_v0, ea_v0, xj_v1, ea_v1,
                    row_v0, row_v1, sidx0, sidx1, w_v, att_v, s0_v, acc_sh,
                    s_si0, s_di0, s_gj0, s_gi0, s_ge0,
                    s_si1, s_di1, s_gj1, s_gi1, s_ge1, s_sc0, s_sc1):
    cid = lax.axis_index("c")
    row_v = row_v0
    sid = lax.axis_index("s")

    bufs = (
        (src_v0, dst_v0, xj_v0, ea_v0, s_si0, s_di0, s_gj0, s_gi0, s_ge0,
         row_v0, sidx0, s_sc0),
        (src_v1, dst_v1, xj_v1, ea_v1, s_si1, s_di1, s_gj1, s_gi1, s_ge1,
         row_v1, sidx1, s_sc1),
    )

    def _zrow(i, carry):
        for cc in range(R // L):
            row_v[i, cc * L:(cc + 1) * L] = jnp.zeros((L,), jnp.float32)
        return carry
    lax.fori_loop(0, B, _zrow, 0)
    row0 = sid * RPT
    def _zcp(j, carry):
        pltpu.sync_copy(row_v, acc_sh.at[pl.ds(row0 + j * B, B)])
        return carry
    lax.fori_loop(0, RPT // B, _zcp, 0)

    def _zs(i, carry):
        s0_v[pl.ds(i * L, L)] = jnp.zeros((L,), jnp.float32)
        return carry
    lax.fori_loop(0, NP // L, _zs, 0)
    plsc.subcore_barrier()

    pltpu.sync_copy(att_hbm, att_v)
    att_c = [att_v[cc * L:(cc + 1) * L] for cc in range(C // L)]
    lane = lax.iota(jnp.int32, L)
    lane0 = lane == 0
    bfly = [jnp.bitwise_xor(lane, sh).reshape(L, 1) for sh in (8, 4, 2, 1)]
    gd = lax.GatherDimensionNumbers(
        offset_dims=(), collapsed_slice_dims=(0,), start_index_map=(0,))

    def _xl_sum(v):
        for idx in bfly:
            v = v + lax.gather(v, idx, gd, slice_sizes=(1,),
                               mode=lax.GatherScatterMode.PROMISE_IN_BOUNDS)
        return v

    ebase = (cid * NS + sid) * EPT

    def _issue_idx(k, b):
        eoff = pl.multiple_of(ebase + k * B, 8)
        pltpu.make_async_copy(src_hbm.at[pl.ds(eoff, B)], b[0], b[4]).start()
        pltpu.make_async_copy(dst_hbm.at[pl.ds(eoff, B)], b[1], b[5]).start()
        pltpu.make_async_copy(ea_hbm.at[pl.ds(eoff, B)], b[3], b[8]).start()

    def _wait_idx(b):
        pltpu.make_async_copy(src_hbm.at[pl.ds(0, B)], b[0], b[4]).wait()
        pltpu.make_async_copy(dst_hbm.at[pl.ds(0, B)], b[1], b[5]).wait()
        pltpu.make_async_copy(ea_hbm.at[pl.ds(0, B)], b[3], b[8]).wait()

    def _issue_data(k, b):
        pltpu.make_async_copy(xl_hbm.at[b[0]], b[2], b[6]).start()
        pltpu.make_async_copy(xr_hbm.at[b[1]], b[3], b[7]).start(add=True)

    def _wait_data(b):
        pltpu.make_async_copy(xl_hbm.at[b[0]], b[2], b[6]).wait()
        pltpu.make_async_copy(xr_hbm.at[b[1]], b[3], b[7]).wait()

    def _compute_scatter(b, first):
        dst_b, xj_b, ea_b = b[1], b[2], b[3]
        rv, sidx, ssc = b[9], b[10], b[11]

        @pl.when(jnp.logical_not(first))
        def _():
            pltpu.make_async_copy(rv, acc_sh.at[sidx], ssc).wait()

        def _edge(e, ecarry):
            acc = jnp.zeros((L,), jnp.float32)
            xjs = []
            for cc in range(C // L):
                vj = xj_b[e, cc * L:(cc + 1) * L]
                ve = ea_b[e, cc * L:(cc + 1) * L]
                v = vj + ve
                v = jnp.where(v > 0, v, SLOPE * v)
                acc = acc + v * att_c[cc]
                xjs.append(vj)
            w = jnp.exp(_xl_sum(acc))
            for cc in range(C // L):
                rv[e, cc * L:(cc + 1) * L] = xjs[cc] * w
            plsc.store_scatter(w_v, [jnp.full((L,), e, jnp.int32)], w,
                               mask=lane0)
            return ecarry
        lax.fori_loop(0, B, _edge, 0, unroll=2)

        for g in range((B + L - 1) // L):
            off = min(g * L, B - L)
            wv = w_v[off:off + L]
            dv = dst_b[off:off + L]
            if off == g * L:
                plsc.addupdate_scatter(s0_v, [dv], wv)
            else:
                plsc.addupdate_scatter(s0_v, [dv], wv,
                                       mask=lane >= (g * L - off))

        for g in (0, L, B - L):
            sidx[g:g + L] = dst_b[g:g + L]
        pltpu.make_async_copy(rv, acc_sh.at[sidx], ssc).start(add=True)

    _issue_idx(0, bufs[0])
    _issue_idx(1, bufs[1])
    _wait_idx(bufs[0])
    _issue_data(0, bufs[0])

    def _body(k2, carry):
        for slot in (0, 1):
            k = k2 * 2 + slot
            b = bufs[slot]
            o = bufs[1 - slot]

            @pl.when(k < NCHUNK - 1)
            def _():
                _wait_idx(o)
                _issue_data(k + 1, o)

            _wait_data(b)
            _compute_scatter(b, k2 == 0)

            @pl.when(k < NCHUNK - 2)
            def _():
                _issue_idx(k + 2, b)
        return carry
    lax.fori_loop(0, NCHUNK // 2, _body, 0)

    pltpu.make_async_copy(row_v0, acc_sh.at[sidx0], s_sc0).wait()
    pltpu.make_async_copy(row_v1, acc_sh.at[sidx1], s_sc1).wait()

    pltpu.sync_copy(s0_v, s0_hbm.at[cid, sid])

    plsc.subcore_barrier()
    def _wb(j, carry):
        r0 = row0 + j * B
        pltpu.sync_copy(acc_sh.at[pl.ds(r0, B)],
                        out_hbm.at[cid, pl.ds(r0, B)])
        return carry
    lax.fori_loop(0, RPT // B, _wb, 0)


@functools.partial(
    pl.kernel,
    mesh=plsc.VectorSubcoreMesh(core_axis_name="c", subcore_axis_name="s"),
    compiler_params=pltpu.CompilerParams(needs_layout_passes=False),
    out_type=[
        jax.ShapeDtypeStruct((NC, NP, R), jnp.float32),
        jax.ShapeDtypeStruct((NC, NS, NP), jnp.float32),
    ],
    scratch_types=[
        pltpu.VMEM((B,), jnp.int32),
        pltpu.VMEM((B,), jnp.int32),
        pltpu.VMEM((B,), jnp.int32),
        pltpu.VMEM((B,), jnp.int32),
        pltpu.VMEM((B, C), jnp.float32),
        pltpu.VMEM((B, C), jnp.float32),
        pltpu.VMEM((B, C), jnp.float32),
        pltpu.VMEM((B, C), jnp.float32),
        pltpu.VMEM((B, C), jnp.float32),
        pltpu.VMEM((B, C), jnp.float32),
        pltpu.VMEM((B,), jnp.int32),
        pltpu.VMEM((B,), jnp.int32),
        pltpu.VMEM((B,), jnp.float32),
        pltpu.VMEM((C,), jnp.float32),
        pltpu.VMEM((NP,), jnp.float32),
        pltpu.VMEM_SHARED((NP, R), jnp.float32),
    ] + [pltpu.SemaphoreType.DMA] * 12,
)
def _edge_pass(src_hbm, dst_hbm, xl_hbm, xr_hbm, ea_hbm, att_hbm, out_hbm,
               s0_hbm,
               src_v0, dst_v0, src_v1, dst_v1,
               xj_v0, ea_v0, xj_v1, ea_v1,
               row_v0, row_v1, sidx0, sidx1, w_v, att_v, s0_v, acc_sh,
               s_si0, s_di0, s_gj0, s_gi0, s_ge0,
               s_si1, s_di1, s_gj1, s_gi1, s_ge1, s_sc0, s_sc1):
    _edge_pass_body(src_hbm, dst_hbm, xl_hbm, xr_hbm, ea_hbm, att_hbm,
                    out_hbm, s0_hbm,
                    src_v0, dst_v0, src_v1, dst_v1,
                    xj_v0, ea_v0, xj_v1, ea_v1,
                    row_v0, row_v1, sidx0, sidx1, w_v, att_v, s0_v, acc_sh,
                    s_si0, s_di0, s_gj0, s_gi0, s_ge0,
                    s_si1, s_di1, s_gj1, s_gi1, s_ge1, s_sc0, s_sc1)


def kernel(x, edge_index, edge_attr, Wl1, bl1, Wr1, br1, We1, att1, bo1,
           Wl2, bl2, Wr2, br2, We2, att2, bo2, Wd1, bd1, Wd2, bd2):
    src = edge_index[0]
    dst = edge_index[1]

    xl1, xr1 = _node_xfm(x, Wl1, bl1, Wr1, br1)
    eaw1, eaw2 = _eaw(edge_attr, We1, We2)
    acc1, s01 = _edge_pass(src, dst, xl1, xr1, eaw1, att1.reshape(C))

    xl2, xr2 = _head(acc1, s01, bo1, Wl2, bl2, Wr2, br2)
    acc2, s02 = _edge_pass(src, dst, xl2, xr2, eaw2, att2.reshape(C))

    return _final(acc2, s02, bo2, Wd1, bd1, Wd2, bd2)[:N]


# speedup vs baseline: 1.0664x; 1.0110x over previous
import functools

import jax
import jax.numpy as jnp
from jax import lax
from jax.experimental import pallas as pl
from jax.experimental.pallas import tpu as pltpu
from jax.experimental.pallas import tpu_sc as plsc

N = 10000
E = 320000
C = 128
ED = 16
DD = 128
DO = 2
SLOPE = 0.2
NL = 0.01

NC = 2
NS = 16
L = 16

R = C
NP = 10240
EPT = E // (NC * NS)
B = 40
RPT = NP // NS


def _node_xfm_body(x_ref, wl_ref, bl_ref, wr_ref, br_ref, xl_ref, xr_ref):
    xb = x_ref[...]
    xl_ref[...] = jnp.dot(xb, wl_ref[...], preferred_element_type=jnp.float32) + bl_ref[...]
    xr_ref[...] = jnp.dot(xb, wr_ref[...], preferred_element_type=jnp.float32) + br_ref[...]


def _node_xfm(x, Wl, bl, Wr, br):
    bn = 1000
    grid = (N // bn,)
    return pl.pallas_call(
        _node_xfm_body,
        grid=grid,
        in_specs=[
            pl.BlockSpec((bn, C), lambda i: (i, 0)),
            pl.BlockSpec((C, C), lambda i: (0, 0)),
            pl.BlockSpec((1, C), lambda i: (0, 0)),
            pl.BlockSpec((C, C), lambda i: (0, 0)),
            pl.BlockSpec((1, C), lambda i: (0, 0)),
        ],
        out_specs=[
            pl.BlockSpec((bn, C), lambda i: (i, 0)),
            pl.BlockSpec((bn, C), lambda i: (i, 0)),
        ],
        out_shape=[
            jax.ShapeDtypeStruct((N, C), jnp.float32),
            jax.ShapeDtypeStruct((N, C), jnp.float32),
        ],
    )(x, Wl, bl.reshape(1, C), Wr, br.reshape(1, C))


def _eaw_body(ea_ref, we1_ref, we2_ref, out1_ref, out2_ref):
    eb = ea_ref[...]
    out1_ref[...] = jnp.dot(eb, we1_ref[...], preferred_element_type=jnp.float32)
    out2_ref[...] = jnp.dot(eb, we2_ref[...], preferred_element_type=jnp.float32)


def _eaw(edge_attr, We1, We2):
    be = 4000
    grid = (E // be,)
    return pl.pallas_call(
        _eaw_body,
        grid=grid,
        in_specs=[
            pl.BlockSpec((be, ED), lambda i: (i, 0)),
            pl.BlockSpec((ED, C), lambda i: (0, 0)),
            pl.BlockSpec((ED, C), lambda i: (0, 0)),
        ],
        out_specs=[
            pl.BlockSpec((be, C), lambda i: (i, 0)),
            pl.BlockSpec((be, C), lambda i: (i, 0)),
        ],
        out_shape=[
            jax.ShapeDtypeStruct((E, C), jnp.float32),
            jax.ShapeDtypeStruct((E, C), jnp.float32),
        ],
    )(edge_attr, We1, We2)


def _norm_h(acc_ref, s0_ref, bo_ref):
    acc_blk = acc_ref[...]
    num = acc_blk[0] + acc_blk[1]
    s = jnp.sum(s0_ref[...], axis=(0, 1))
    h = num / (s + 1e-16) + bo_ref[...]
    return jnp.where(h > 0, h, NL * h)


def _head_body(acc_ref, s0_ref, bo_ref, wl_ref, bl_ref, wr_ref, br_ref, xl_ref, xr_ref):
    h = _norm_h(acc_ref, s0_ref, bo_ref)
    xl_ref[...] = jnp.dot(h, wl_ref[...], preferred_element_type=jnp.float32) + bl_ref[...]
    xr_ref[...] = jnp.dot(h, wr_ref[...], preferred_element_type=jnp.float32) + br_ref[...]


def _head(acc, s0, bo, Wl, bl, Wr, br):
    bn = 1024
    grid = (NP // bn,)
    return pl.pallas_call(
        _head_body,
        grid=grid,
        in_specs=[
            pl.BlockSpec((NC, bn, C), lambda i: (0, i, 0)),
            pl.BlockSpec((NC, NS, bn, 1), lambda i: (0, 0, i, 0)),
            pl.BlockSpec((1, C), lambda i: (0, 0)),
            pl.BlockSpec((C, C), lambda i: (0, 0)),
            pl.BlockSpec((1, C), lambda i: (0, 0)),
            pl.BlockSpec((C, C), lambda i: (0, 0)),
            pl.BlockSpec((1, C), lambda i: (0, 0)),
        ],
        out_specs=[
            pl.BlockSpec((bn, C), lambda i: (i, 0)),
            pl.BlockSpec((bn, C), lambda i: (i, 0)),
        ],
        out_shape=[
            jax.ShapeDtypeStruct((NP, C), jnp.float32),
            jax.ShapeDtypeStruct((NP, C), jnp.float32),
        ],
    )(acc, s0.reshape(NC, NS, NP, 1), bo.reshape(1, C), Wl, bl.reshape(1, C),
      Wr, br.reshape(1, C))


def _final_body(acc_ref, s0_ref, bo_ref, wd1_ref, bd1_ref, wd2_ref, bd2_ref,
                out_ref):
    h = _norm_h(acc_ref, s0_ref, bo_ref)
    d = jnp.dot(h, wd1_ref[...], preferred_element_type=jnp.float32) + bd1_ref[...]
    d = jnp.where(d > 0, d, NL * d)
    out_ref[...] = jnp.dot(d, wd2_ref[...], preferred_element_type=jnp.float32) + bd2_ref[...]


def _final(acc, s0, bo, Wd1, bd1, Wd2, bd2):
    bn = 1024
    grid = (NP // bn,)
    return pl.pallas_call(
        _final_body,
        grid=grid,
        in_specs=[
            pl.BlockSpec((NC, bn, C), lambda i: (0, i, 0)),
            pl.BlockSpec((NC, NS, bn, 1), lambda i: (0, 0, i, 0)),
            pl.BlockSpec((1, C), lambda i: (0, 0)),
            pl.BlockSpec((C, DD), lambda i: (0, 0)),
            pl.BlockSpec((1, DD), lambda i: (0, 0)),
            pl.BlockSpec((DD, DO), lambda i: (0, 0)),
            pl.BlockSpec((1, DO), lambda i: (0, 0)),
        ],
        out_specs=pl.BlockSpec((bn, DO), lambda i: (i, 0)),
        out_shape=jax.ShapeDtypeStruct((NP, DO), jnp.float32),
    )(acc, s0.reshape(NC, NS, NP, 1), bo.reshape(1, C), Wd1,
      bd1.reshape(1, DD), Wd2, bd2.reshape(1, DO))


NCHUNK = EPT // B


def _edge_pass_body(src_hbm, dst_hbm, xl_hbm, xr_hbm, ea_hbm, att_hbm,
                    out_hbm, s0_hbm,
                    src_v0, dst_v0, src_v1, dst_v1,
                    xj_v0, ea_v0, xj_v1, ea_v1,
                    row_v0, row_v1, sidx0, sidx1, w_v, att_v, s0_v, acc_sh,
                    s_si0, s_di0, s_gj0, s_gi0, s_ge0,
                    s_si1, s_di1, s_gj1, s_gi1, s_ge1, s_sc0, s_sc1):
    cid = lax.axis_index("c")
    row_v = row_v0
    sid = lax.axis_index("s")

    bufs = (
        (src_v0, dst_v0, xj_v0, ea_v0, s_si0, s_di0, s_gj0, s_gi0, s_ge0,
         row_v0, sidx0, s_sc0),
        (src_v1, dst_v1, xj_v1, ea_v1, s_si1, s_di1, s_gj1, s_gi1, s_ge1,
         row_v1, sidx1, s_sc1),
    )

    def _zrow(i, carry):
        for cc in range(R // L):
            row_v[i, cc * L:(cc + 1) * L] = jnp.zeros((L,), jnp.float32)
        return carry
    lax.fori_loop(0, B, _zrow, 0)
    row0 = sid * RPT
    def _zcp(j, carry):
        pltpu.sync_copy(row_v, acc_sh.at[pl.ds(row0 + j * B, B)])
        return carry
    lax.fori_loop(0, RPT // B, _zcp, 0)

    def _zs(i, carry):
        s0_v[pl.ds(i * L, L)] = jnp.zeros((L,), jnp.float32)
        return carry
    lax.fori_loop(0, NP // L, _zs, 0)
    plsc.subcore_barrier()

    pltpu.sync_copy(att_hbm, att_v)
    att_c = [att_v[cc * L:(cc + 1) * L] for cc in range(C // L)]
    lane = lax.iota(jnp.int32, L)
    lane0 = lane == 0
    bfly = [jnp.bitwise_xor(lane, sh).reshape(L, 1) for sh in (8, 4, 2, 1)]
    gd = lax.GatherDimensionNumbers(
        offset_dims=(), collapsed_slice_dims=(0,), start_index_map=(0,))

    def _xl_sum(v):
        for idx in bfly:
            v = v + lax.gather(v, idx, gd, slice_sizes=(1,),
                               mode=lax.GatherScatterMode.PROMISE_IN_BOUNDS)
        return v

    ebase = (cid * NS + sid) * EPT

    def _issue_idx(k, b):
        eoff = pl.multiple_of(ebase + k * B, 8)
        pltpu.make_async_copy(src_hbm.at[pl.ds(eoff, B)], b[0], b[4]).start()
        pltpu.make_async_copy(dst_hbm.at[pl.ds(eoff, B)], b[1], b[5]).start()
        pltpu.make_async_copy(ea_hbm.at[pl.ds(eoff, B)], b[3], b[8]).start()

    def _wait_idx(b):
        pltpu.make_async_copy(src_hbm.at[pl.ds(0, B)], b[0], b[4]).wait()
        pltpu.make_async_copy(dst_hbm.at[pl.ds(0, B)], b[1], b[5]).wait()
        pltpu.make_async_copy(ea_hbm.at[pl.ds(0, B)], b[3], b[8]).wait()

    def _issue_data(k, b):
        pltpu.make_async_copy(xl_hbm.at[b[0]], b[2], b[6]).start()
        pltpu.make_async_copy(xr_hbm.at[b[1]], b[3], b[7]).start(add=True)

    def _wait_data(b):
        pltpu.make_async_copy(xl_hbm.at[b[0]], b[2], b[6]).wait()
        pltpu.make_async_copy(xr_hbm.at[b[1]], b[3], b[7]).wait()

    def _compute_scatter(b, first):
        dst_b, xj_b, ea_b = b[1], b[2], b[3]
        rv, sidx, ssc = b[9], b[10], b[11]

        @pl.when(jnp.logical_not(first))
        def _():
            pltpu.make_async_copy(rv, acc_sh.at[sidx], ssc).wait()

        def _edge(e, ecarry):
            acc = jnp.zeros((L,), jnp.float32)
            xjs = []
            for cc in range(C // L):
                vj = xj_b[e, cc * L:(cc + 1) * L]
                ve = ea_b[e, cc * L:(cc + 1) * L]
                v = vj + ve
                v = jnp.where(v > 0, v, SLOPE * v)
                acc = acc + v * att_c[cc]
                xjs.append(vj)
            w = jnp.exp(_xl_sum(acc))
            for cc in range(C // L):
                rv[e, cc * L:(cc + 1) * L] = xjs[cc] * w
            plsc.store_scatter(w_v, [jnp.full((L,), e, jnp.int32)], w,
                               mask=lane0)
            return ecarry
        lax.fori_loop(0, B, _edge, 0)

        for g in range((B + L - 1) // L):
            off = min(g * L, B - L)
            wv = w_v[off:off + L]
            dv = dst_b[off:off + L]
            if off == g * L:
                plsc.addupdate_scatter(s0_v, [dv], wv)
            else:
                plsc.addupdate_scatter(s0_v, [dv], wv,
                                       mask=lane >= (g * L - off))

        for g in (0, L, B - L):
            sidx[g:g + L] = dst_b[g:g + L]
        pltpu.make_async_copy(rv, acc_sh.at[sidx], ssc).start(add=True)

    _issue_idx(0, bufs[0])
    _issue_idx(1, bufs[1])
    _wait_idx(bufs[0])
    _issue_data(0, bufs[0])

    def _body(k2, carry):
        for slot in (0, 1):
            k = k2 * 2 + slot
            b = bufs[slot]
            o = bufs[1 - slot]

            @pl.when(k < NCHUNK - 1)
            def _():
                _wait_idx(o)
                _issue_data(k + 1, o)

            _wait_data(b)
            _compute_scatter(b, k2 == 0)

            @pl.when(k < NCHUNK - 2)
            def _():
                _issue_idx(k + 2, b)
        return carry
    lax.fori_loop(0, NCHUNK // 2, _body, 0)

    pltpu.make_async_copy(row_v0, acc_sh.at[sidx0], s_sc0).wait()
    pltpu.make_async_copy(row_v1, acc_sh.at[sidx1], s_sc1).wait()

    pltpu.sync_copy(s0_v, s0_hbm.at[cid, sid])

    plsc.subcore_barrier()
    def _wb(j, carry):
        r0 = row0 + j * B
        pltpu.sync_copy(acc_sh.at[pl.ds(r0, B)],
                        out_hbm.at[cid, pl.ds(r0, B)])
        return carry
    lax.fori_loop(0, RPT // B, _wb, 0)


@functools.partial(
    pl.kernel,
    mesh=plsc.VectorSubcoreMesh(core_axis_name="c", subcore_axis_name="s"),
    compiler_params=pltpu.CompilerParams(needs_layout_passes=False),
    out_type=[
        jax.ShapeDtypeStruct((NC, NP, R), jnp.float32),
        jax.ShapeDtypeStruct((NC, NS, NP), jnp.float32),
    ],
    scratch_types=[
        pltpu.VMEM((B,), jnp.int32),
        pltpu.VMEM((B,), jnp.int32),
        pltpu.VMEM((B,), jnp.int32),
        pltpu.VMEM((B,), jnp.int32),
        pltpu.VMEM((B, C), jnp.float32),
        pltpu.VMEM((B, C), jnp.float32),
        pltpu.VMEM((B, C), jnp.float32),
        pltpu.VMEM((B, C), jnp.float32),
        pltpu.VMEM((B, C), jnp.float32),
        pltpu.VMEM((B, C), jnp.float32),
        pltpu.VMEM((B,), jnp.int32),
        pltpu.VMEM((B,), jnp.int32),
        pltpu.VMEM((B,), jnp.float32),
        pltpu.VMEM((C,), jnp.float32),
        pltpu.VMEM((NP,), jnp.float32),
        pltpu.VMEM_SHARED((NP, R), jnp.float32),
    ] + [pltpu.SemaphoreType.DMA] * 12,
)
def _edge_pass(src_hbm, dst_hbm, xl_hbm, xr_hbm, ea_hbm, att_hbm, out_hbm,
               s0_hbm,
               src_v0, dst_v0, src_v1, dst_v1,
               xj_v0, ea_v0, xj_v1, ea_v1,
               row_v0, row_v1, sidx0, sidx1, w_v, att_v, s0_v, acc_sh,
               s_si0, s_di0, s_gj0, s_gi0, s_ge0,
               s_si1, s_di1, s_gj1, s_gi1, s_ge1, s_sc0, s_sc1):
    _edge_pass_body(src_hbm, dst_hbm, xl_hbm, xr_hbm, ea_hbm, att_hbm,
                    out_hbm, s0_hbm,
                    src_v0, dst_v0, src_v1, dst_v1,
                    xj_v0, ea_v0, xj_v1, ea_v1,
                    row_v0, row_v1, sidx0, sidx1, w_v, att_v, s0_v, acc_sh,
                    s_si0, s_di0, s_gj0, s_gi0, s_ge0,
                    s_si1, s_di1, s_gj1, s_gi1, s_ge1, s_sc0, s_sc1)


def kernel(x, edge_index, edge_attr, Wl1, bl1, Wr1, br1, We1, att1, bo1,
           Wl2, bl2, Wr2, br2, We2, att2, bo2, Wd1, bd1, Wd2, bd2):
    src = edge_index[0]
    dst = edge_index[1]

    xl1, xr1 = _node_xfm(x, Wl1, bl1, Wr1, br1)
    eaw1, eaw2 = _eaw(edge_attr, We1, We2)
    acc1, s01 = _edge_pass(src, dst, xl1, xr1, eaw1, att1.reshape(C))

    xl2, xr2 = _head(acc1, s01, bo1, Wl2, bl2, Wr2, br2)
    acc2, s02 = _edge_pass(src, dst, xl2, xr2, eaw2, att2.reshape(C))

    return _final(acc2, s02, bo2, Wd1, bd1, Wd2, bd2)[:N]


# speedup vs baseline: 1.0727x; 1.0060x over previous
import functools

import jax
import jax.numpy as jnp
from jax import lax
from jax.experimental import pallas as pl
from jax.experimental.pallas import tpu as pltpu
from jax.experimental.pallas import tpu_sc as plsc

N = 10000
E = 320000
C = 128
ED = 16
DD = 128
DO = 2
SLOPE = 0.2
NL = 0.01

NC = 2
NS = 16
L = 16

R = C
NP = 10240
EPT = E // (NC * NS)
B = 40
RPT = NP // NS


def _node_xfm_body(x_ref, wl_ref, bl_ref, wr_ref, br_ref, xl_ref, xr_ref):
    xb = x_ref[...]
    xl_ref[...] = jnp.dot(xb, wl_ref[...], preferred_element_type=jnp.float32) + bl_ref[...]
    xr_ref[...] = jnp.dot(xb, wr_ref[...], preferred_element_type=jnp.float32) + br_ref[...]


def _node_xfm(x, Wl, bl, Wr, br):
    bn = 1000
    grid = (N // bn,)
    return pl.pallas_call(
        _node_xfm_body,
        grid=grid,
        in_specs=[
            pl.BlockSpec((bn, C), lambda i: (i, 0)),
            pl.BlockSpec((C, C), lambda i: (0, 0)),
            pl.BlockSpec((1, C), lambda i: (0, 0)),
            pl.BlockSpec((C, C), lambda i: (0, 0)),
            pl.BlockSpec((1, C), lambda i: (0, 0)),
        ],
        out_specs=[
            pl.BlockSpec((bn, C), lambda i: (i, 0)),
            pl.BlockSpec((bn, C), lambda i: (i, 0)),
        ],
        out_shape=[
            jax.ShapeDtypeStruct((N, C), jnp.float32),
            jax.ShapeDtypeStruct((N, C), jnp.float32),
        ],
    )(x, Wl, bl.reshape(1, C), Wr, br.reshape(1, C))


def _eaw_body(ea_ref, we1_ref, we2_ref, out1_ref, out2_ref):
    eb = ea_ref[...]
    out1_ref[...] = jnp.dot(eb, we1_ref[...], preferred_element_type=jnp.float32)
    out2_ref[...] = jnp.dot(eb, we2_ref[...], preferred_element_type=jnp.float32)


def _eaw(edge_attr, We1, We2):
    be = 4000
    grid = (E // be,)
    return pl.pallas_call(
        _eaw_body,
        grid=grid,
        in_specs=[
            pl.BlockSpec((be, ED), lambda i: (i, 0)),
            pl.BlockSpec((ED, C), lambda i: (0, 0)),
            pl.BlockSpec((ED, C), lambda i: (0, 0)),
        ],
        out_specs=[
            pl.BlockSpec((be, C), lambda i: (i, 0)),
            pl.BlockSpec((be, C), lambda i: (i, 0)),
        ],
        out_shape=[
            jax.ShapeDtypeStruct((E, C), jnp.float32),
            jax.ShapeDtypeStruct((E, C), jnp.float32),
        ],
    )(edge_attr, We1, We2)


def _norm_h(acc_ref, s0_ref, bo_ref):
    acc_blk = acc_ref[...]
    num = acc_blk[0] + acc_blk[1]
    s = jnp.sum(s0_ref[...], axis=(0, 1))
    h = num / (s + 1e-16) + bo_ref[...]
    return jnp.where(h > 0, h, NL * h)


def _head_body(acc_ref, s0_ref, bo_ref, wl_ref, bl_ref, wr_ref, br_ref, xl_ref, xr_ref):
    h = _norm_h(acc_ref, s0_ref, bo_ref)
    xl_ref[...] = jnp.dot(h, wl_ref[...], preferred_element_type=jnp.float32) + bl_ref[...]
    xr_ref[...] = jnp.dot(h, wr_ref[...], preferred_element_type=jnp.float32) + br_ref[...]


def _head(acc, s0, bo, Wl, bl, Wr, br):
    bn = 1024
    grid = (NP // bn,)
    return pl.pallas_call(
        _head_body,
        grid=grid,
        in_specs=[
            pl.BlockSpec((NC, bn, C), lambda i: (0, i, 0)),
            pl.BlockSpec((NC, NS, bn, 1), lambda i: (0, 0, i, 0)),
            pl.BlockSpec((1, C), lambda i: (0, 0)),
            pl.BlockSpec((C, C), lambda i: (0, 0)),
            pl.BlockSpec((1, C), lambda i: (0, 0)),
            pl.BlockSpec((C, C), lambda i: (0, 0)),
            pl.BlockSpec((1, C), lambda i: (0, 0)),
        ],
        out_specs=[
            pl.BlockSpec((bn, C), lambda i: (i, 0)),
            pl.BlockSpec((bn, C), lambda i: (i, 0)),
        ],
        out_shape=[
            jax.ShapeDtypeStruct((NP, C), jnp.float32),
            jax.ShapeDtypeStruct((NP, C), jnp.float32),
        ],
    )(acc, s0.reshape(NC, NS, NP, 1), bo.reshape(1, C), Wl, bl.reshape(1, C),
      Wr, br.reshape(1, C))


def _final_body(acc_ref, s0_ref, bo_ref, wd1_ref, bd1_ref, wd2_ref, bd2_ref,
                out_ref):
    h = _norm_h(acc_ref, s0_ref, bo_ref)
    d = jnp.dot(h, wd1_ref[...], preferred_element_type=jnp.float32) + bd1_ref[...]
    d = jnp.where(d > 0, d, NL * d)
    out_ref[...] = jnp.dot(d, wd2_ref[...], preferred_element_type=jnp.float32) + bd2_ref[...]


def _final(acc, s0, bo, Wd1, bd1, Wd2, bd2):
    bn = 1024
    grid = (NP // bn,)
    return pl.pallas_call(
        _final_body,
        grid=grid,
        in_specs=[
            pl.BlockSpec((NC, bn, C), lambda i: (0, i, 0)),
            pl.BlockSpec((NC, NS, bn, 1), lambda i: (0, 0, i, 0)),
            pl.BlockSpec((1, C), lambda i: (0, 0)),
            pl.BlockSpec((C, DD), lambda i: (0, 0)),
            pl.BlockSpec((1, DD), lambda i: (0, 0)),
            pl.BlockSpec((DD, DO), lambda i: (0, 0)),
            pl.BlockSpec((1, DO), lambda i: (0, 0)),
        ],
        out_specs=pl.BlockSpec((bn, DO), lambda i: (i, 0)),
        out_shape=jax.ShapeDtypeStruct((NP, DO), jnp.float32),
    )(acc, s0.reshape(NC, NS, NP, 1), bo.reshape(1, C), Wd1,
      bd1.reshape(1, DD), Wd2, bd2.reshape(1, DO))


NCHUNK = EPT // B


def _edge_pass_body(src_hbm, dst_hbm, xl_hbm, xr_hbm, ea_hbm, att_hbm,
                    out_hbm, s0_hbm,
                    src_v0, dst_v0, src_v1, dst_v1,
                    xj_v0, ea_v0, xj_v1, ea_v1,
                    row_v0, row_v1, sidx0, sidx1, w_v, att_v, s0_v, acc_sh,
                    s_si0, s_di0, s_gj0, s_gi0, s_ge0,
                    s_si1, s_di1, s_gj1, s_gi1, s_ge1, s_sc0, s_sc1):
    cid = lax.axis_index("c")
    row_v = row_v0
    sid = lax.axis_index("s")

    bufs = (
        (src_v0, dst_v0, xj_v0, ea_v0, s_si0, s_di0, s_gj0, s_gi0, s_ge0,
         row_v0, sidx0, s_sc0),
        (src_v1, dst_v1, xj_v1, ea_v1, s_si1, s_di1, s_gj1, s_gi1, s_ge1,
         row_v1, sidx1, s_sc1),
    )

    def _zrow(i, carry):
        for cc in range(R // L):
            row_v[i, cc * L:(cc + 1) * L] = jnp.zeros((L,), jnp.float32)
        return carry
    lax.fori_loop(0, B, _zrow, 0)
    row0 = sid * RPT
    for j in range(RPT // B):
        pltpu.make_async_copy(row_v, acc_sh.at[pl.ds(row0 + j * B, B)],
                              s_sc0).start()
    for j in range(RPT // B):
        pltpu.make_async_copy(row_v, acc_sh.at[pl.ds(row0, B)],
                              s_sc0).wait()

    def _zs(i, carry):
        s0_v[pl.ds(i * L, L)] = jnp.zeros((L,), jnp.float32)
        return carry
    lax.fori_loop(0, NP // L, _zs, 0)
    plsc.subcore_barrier()

    pltpu.sync_copy(att_hbm, att_v)
    att_c = [att_v[cc * L:(cc + 1) * L] for cc in range(C // L)]
    lane = lax.iota(jnp.int32, L)
    lane0 = lane == 0
    bfly = [jnp.bitwise_xor(lane, sh).reshape(L, 1) for sh in (8, 4, 2, 1)]
    gd = lax.GatherDimensionNumbers(
        offset_dims=(), collapsed_slice_dims=(0,), start_index_map=(0,))

    def _xl_sum(v):
        for idx in bfly:
            v = v + lax.gather(v, idx, gd, slice_sizes=(1,),
                               mode=lax.GatherScatterMode.PROMISE_IN_BOUNDS)
        return v

    ebase = (cid * NS + sid) * EPT

    def _issue_idx(k, b):
        eoff = pl.multiple_of(ebase + k * B, 8)
        pltpu.make_async_copy(src_hbm.at[pl.ds(eoff, B)], b[0], b[4]).start()
        pltpu.make_async_copy(dst_hbm.at[pl.ds(eoff, B)], b[1], b[5]).start()
        pltpu.make_async_copy(ea_hbm.at[pl.ds(eoff, B)], b[3], b[8]).start()

    def _wait_idx(b):
        pltpu.make_async_copy(src_hbm.at[pl.ds(0, B)], b[0], b[4]).wait()
        pltpu.make_async_copy(dst_hbm.at[pl.ds(0, B)], b[1], b[5]).wait()
        pltpu.make_async_copy(ea_hbm.at[pl.ds(0, B)], b[3], b[8]).wait()

    def _issue_data(k, b):
        pltpu.make_async_copy(xl_hbm.at[b[0]], b[2], b[6]).start()
        pltpu.make_async_copy(xr_hbm.at[b[1]], b[3], b[7]).start(add=True)

    def _wait_data(b):
        pltpu.make_async_copy(xl_hbm.at[b[0]], b[2], b[6]).wait()
        pltpu.make_async_copy(xr_hbm.at[b[1]], b[3], b[7]).wait()

    def _compute_scatter(b, first):
        dst_b, xj_b, ea_b = b[1], b[2], b[3]
        rv, sidx, ssc = b[9], b[10], b[11]

        @pl.when(jnp.logical_not(first))
        def _():
            pltpu.make_async_copy(rv, acc_sh.at[sidx], ssc).wait()

        def _edge(e, ecarry):
            acc = jnp.zeros((L,), jnp.float32)
            xjs = []
            for cc in range(C // L):
                vj = xj_b[e, cc * L:(cc + 1) * L]
                ve = ea_b[e, cc * L:(cc + 1) * L]
                v = vj + ve
                v = jnp.where(v > 0, v, SLOPE * v)
                acc = acc + v * att_c[cc]
                xjs.append(vj)
            w = jnp.exp(_xl_sum(acc))
            for cc in range(C // L):
                rv[e, cc * L:(cc + 1) * L] = xjs[cc] * w
            plsc.store_scatter(w_v, [jnp.full((L,), e, jnp.int32)], w,
                               mask=lane0)
            return ecarry
        lax.fori_loop(0, B, _edge, 0)

        for g in range((B + L - 1) // L):
            off = min(g * L, B - L)
            wv = w_v[off:off + L]
            dv = dst_b[off:off + L]
            if off == g * L:
                plsc.addupdate_scatter(s0_v, [dv], wv)
            else:
                plsc.addupdate_scatter(s0_v, [dv], wv,
                                       mask=lane >= (g * L - off))

        for g in (0, L, B - L):
            sidx[g:g + L] = dst_b[g:g + L]
        pltpu.make_async_copy(rv, acc_sh.at[sidx], ssc).start(add=True)

    _issue_idx(0, bufs[0])
    _issue_idx(1, bufs[1])
    _wait_idx(bufs[0])
    _issue_data(0, bufs[0])

    def _body(k2, carry):
        for slot in (0, 1):
            k = k2 * 2 + slot
            b = bufs[slot]
            o = bufs[1 - slot]

            @pl.when(k < NCHUNK - 1)
            def _():
                _wait_idx(o)
                _issue_data(k + 1, o)

            _wait_data(b)
            _compute_scatter(b, k2 == 0)

            @pl.when(k < NCHUNK - 2)
            def _():
                _issue_idx(k + 2, b)
        return carry
    lax.fori_loop(0, NCHUNK // 2, _body, 0)

    pltpu.make_async_copy(row_v0, acc_sh.at[sidx0], s_sc0).wait()
    pltpu.make_async_copy(row_v1, acc_sh.at[sidx1], s_sc1).wait()

    pltpu.sync_copy(s0_v, s0_hbm.at[cid, sid])

    plsc.subcore_barrier()
    pltpu.sync_copy(acc_sh.at[pl.ds(row0, RPT)],
                    out_hbm.at[cid, pl.ds(row0, RPT)])


@functools.partial(
    pl.kernel,
    mesh=plsc.VectorSubcoreMesh(core_axis_name="c", subcore_axis_name="s"),
    compiler_params=pltpu.CompilerParams(needs_layout_passes=False),
    out_type=[
        jax.ShapeDtypeStruct((NC, NP, R), jnp.float32),
        jax.ShapeDtypeStruct((NC, NS, NP), jnp.float32),
    ],
    scratch_types=[
        pltpu.VMEM((B,), jnp.int32),
        pltpu.VMEM((B,), jnp.int32),
        pltpu.VMEM((B,), jnp.int32),
        pltpu.VMEM((B,), jnp.int32),
        pltpu.VMEM((B, C), jnp.float32),
        pltpu.VMEM((B, C), jnp.float32),
        pltpu.VMEM((B, C), jnp.float32),
        pltpu.VMEM((B, C), jnp.float32),
        pltpu.VMEM((B, C), jnp.float32),
        pltpu.VMEM((B, C), jnp.float32),
        pltpu.VMEM((B,), jnp.int32),
        pltpu.VMEM((B,), jnp.int32),
        pltpu.VMEM((B,), jnp.float32),
        pltpu.VMEM((C,), jnp.float32),
        pltpu.VMEM((NP,), jnp.float32),
        pltpu.VMEM_SHARED((NP, R), jnp.float32),
    ] + [pltpu.SemaphoreType.DMA] * 12,
)
def _edge_pass(src_hbm, dst_hbm, xl_hbm, xr_hbm, ea_hbm, att_hbm, out_hbm,
               s0_hbm,
               src_v0, dst_v0, src_v1, dst_v1,
               xj_v0, ea_v0, xj_v1, ea_v1,
               row_v0, row_v1, sidx0, sidx1, w_v, att_v, s0_v, acc_sh,
               s_si0, s_di0, s_gj0, s_gi0, s_ge0,
               s_si1, s_di1, s_gj1, s_gi1, s_ge1, s_sc0, s_sc1):
    _edge_pass_body(src_hbm, dst_hbm, xl_hbm, xr_hbm, ea_hbm, att_hbm,
                    out_hbm, s0_hbm,
                    src_v0, dst_v0, src_v1, dst_v1,
                    xj_v0, ea_v0, xj_v1, ea_v1,
                    row_v0, row_v1, sidx0, sidx1, w_v, att_v, s0_v, acc_sh,
                    s_si0, s_di0, s_gj0, s_gi0, s_ge0,
                    s_si1, s_di1, s_gj1, s_gi1, s_ge1, s_sc0, s_sc1)


def kernel(x, edge_index, edge_attr, Wl1, bl1, Wr1, br1, We1, att1, bo1,
           Wl2, bl2, Wr2, br2, We2, att2, bo2, Wd1, bd1, Wd2, bd2):
    src = edge_index[0]
    dst = edge_index[1]

    xl1, xr1 = _node_xfm(x, Wl1, bl1, Wr1, br1)
    eaw1, eaw2 = _eaw(edge_attr, We1, We2)
    acc1, s01 = _edge_pass(src, dst, xl1, xr1, eaw1, att1.reshape(C))

    xl2, xr2 = _head(acc1, s01, bo1, Wl2, bl2, Wr2, br2)
    acc2, s02 = _edge_pass(src, dst, xl2, xr2, eaw2, att2.reshape(C))

    return _final(acc2, s02, bo2, Wd1, bd1, Wd2, bd2)[:N]


# speedup vs baseline: 1.0806x; 1.0073x over previous
import functools

import jax
import jax.numpy as jnp
from jax import lax
from jax.experimental import pallas as pl
from jax.experimental.pallas import tpu as pltpu
from jax.experimental.pallas import tpu_sc as plsc

N = 10000
E = 320000
C = 128
ED = 16
DD = 128
DO = 2
SLOPE = 0.2
NL = 0.01

NC = 2
NS = 16
L = 16

R = C
NP = 10240
EPT = E // (NC * NS)
B = 40
RPT = NP // NS


def _node_xfm_body(x_ref, wl_ref, bl_ref, wr_ref, br_ref, xl_ref, xr_ref):
    xb = x_ref[...]
    xl_ref[...] = jnp.dot(xb, wl_ref[...], preferred_element_type=jnp.float32) + bl_ref[...]
    xr_ref[...] = jnp.dot(xb, wr_ref[...], preferred_element_type=jnp.float32) + br_ref[...]


def _node_xfm(x, Wl, bl, Wr, br):
    bn = 1000
    grid = (N // bn,)
    return pl.pallas_call(
        _node_xfm_body,
        grid=grid,
        in_specs=[
            pl.BlockSpec((bn, C), lambda i: (i, 0)),
            pl.BlockSpec((C, C), lambda i: (0, 0)),
            pl.BlockSpec((1, C), lambda i: (0, 0)),
            pl.BlockSpec((C, C), lambda i: (0, 0)),
            pl.BlockSpec((1, C), lambda i: (0, 0)),
        ],
        out_specs=[
            pl.BlockSpec((bn, C), lambda i: (i, 0)),
            pl.BlockSpec((bn, C), lambda i: (i, 0)),
        ],
        out_shape=[
            jax.ShapeDtypeStruct((N, C), jnp.float32),
            jax.ShapeDtypeStruct((N, C), jnp.float32),
        ],
    )(x, Wl, bl.reshape(1, C), Wr, br.reshape(1, C))


def _eaw_body(ea_ref, we_ref, out_ref):
    out_ref[...] = jnp.dot(ea_ref[...], we_ref[...], preferred_element_type=jnp.float32)


def _eaw(edge_attr, We):
    be = 4000
    grid = (E // be,)
    return pl.pallas_call(
        _eaw_body,
        grid=grid,
        in_specs=[
            pl.BlockSpec((be, ED), lambda i: (i, 0)),
            pl.BlockSpec((ED, C), lambda i: (0, 0)),
        ],
        out_specs=pl.BlockSpec((be, C), lambda i: (i, 0)),
        out_shape=jax.ShapeDtypeStruct((E, C), jnp.float32),
    )(edge_attr, We)


def _norm_h(acc_ref, s0_ref, bo_ref):
    acc_blk = acc_ref[...]
    num = acc_blk[0] + acc_blk[1]
    s = jnp.sum(s0_ref[...], axis=(0, 1))
    h = num / (s + 1e-16) + bo_ref[...]
    return jnp.where(h > 0, h, NL * h)


def _head_body(acc_ref, s0_ref, bo_ref, wl_ref, bl_ref, wr_ref, br_ref, xl_ref, xr_ref):
    h = _norm_h(acc_ref, s0_ref, bo_ref)
    xl_ref[...] = jnp.dot(h, wl_ref[...], preferred_element_type=jnp.float32) + bl_ref[...]
    xr_ref[...] = jnp.dot(h, wr_ref[...], preferred_element_type=jnp.float32) + br_ref[...]


def _head(acc, s0, bo, Wl, bl, Wr, br):
    bn = 1024
    grid = (NP // bn,)
    return pl.pallas_call(
        _head_body,
        grid=grid,
        in_specs=[
            pl.BlockSpec((NC, bn, C), lambda i: (0, i, 0)),
            pl.BlockSpec((NC, NS, bn, 1), lambda i: (0, 0, i, 0)),
            pl.BlockSpec((1, C), lambda i: (0, 0)),
            pl.BlockSpec((C, C), lambda i: (0, 0)),
            pl.BlockSpec((1, C), lambda i: (0, 0)),
            pl.BlockSpec((C, C), lambda i: (0, 0)),
            pl.BlockSpec((1, C), lambda i: (0, 0)),
        ],
        out_specs=[
            pl.BlockSpec((bn, C), lambda i: (i, 0)),
            pl.BlockSpec((bn, C), lambda i: (i, 0)),
        ],
        out_shape=[
            jax.ShapeDtypeStruct((NP, C), jnp.float32),
            jax.ShapeDtypeStruct((NP, C), jnp.float32),
        ],
    )(acc, s0.reshape(NC, NS, NP, 1), bo.reshape(1, C), Wl, bl.reshape(1, C),
      Wr, br.reshape(1, C))


def _final_body(acc_ref, s0_ref, bo_ref, wd1_ref, bd1_ref, wd2_ref, bd2_ref,
                out_ref):
    h = _norm_h(acc_ref, s0_ref, bo_ref)
    d = jnp.dot(h, wd1_ref[...], preferred_element_type=jnp.float32) + bd1_ref[...]
    d = jnp.where(d > 0, d, NL * d)
    out_ref[...] = jnp.dot(d, wd2_ref[...], preferred_element_type=jnp.float32) + bd2_ref[...]


def _final(acc, s0, bo, Wd1, bd1, Wd2, bd2):
    bn = 1024
    grid = (NP // bn,)
    return pl.pallas_call(
        _final_body,
        grid=grid,
        in_specs=[
            pl.BlockSpec((NC, bn, C), lambda i: (0, i, 0)),
            pl.BlockSpec((NC, NS, bn, 1), lambda i: (0, 0, i, 0)),
            pl.BlockSpec((1, C), lambda i: (0, 0)),
            pl.BlockSpec((C, DD), lambda i: (0, 0)),
            pl.BlockSpec((1, DD), lambda i: (0, 0)),
            pl.BlockSpec((DD, DO), lambda i: (0, 0)),
            pl.BlockSpec((1, DO), lambda i: (0, 0)),
        ],
        out_specs=pl.BlockSpec((bn, DO), lambda i: (i, 0)),
        out_shape=jax.ShapeDtypeStruct((NP, DO), jnp.float32),
    )(acc, s0.reshape(NC, NS, NP, 1), bo.reshape(1, C), Wd1,
      bd1.reshape(1, DD), Wd2, bd2.reshape(1, DO))


NCHUNK = EPT // B


def _edge_pass_body(src_hbm, dst_hbm, xl_hbm, xr_hbm, ea_hbm, att_hbm,
                    out_hbm, s0_hbm,
                    src_v0, dst_v0, src_v1, dst_v1,
                    xj_v0, ea_v0, xj_v1, ea_v1,
                    row_v0, row_v1, sidx0, sidx1, w_v, att_v, s0_v, acc_sh,
                    s_si0, s_di0, s_gj0, s_gi0, s_ge0,
                    s_si1, s_di1, s_gj1, s_gi1, s_ge1, s_sc0, s_sc1):
    cid = lax.axis_index("c")
    row_v = row_v0
    sid = lax.axis_index("s")

    bufs = (
        (src_v0, dst_v0, xj_v0, ea_v0, s_si0, s_di0, s_gj0, s_gi0, s_ge0,
         row_v0, sidx0, s_sc0),
        (src_v1, dst_v1, xj_v1, ea_v1, s_si1, s_di1, s_gj1, s_gi1, s_ge1,
         row_v1, sidx1, s_sc1),
    )

    def _zrow(i, carry):
        for cc in range(R // L):
            row_v[i, cc * L:(cc + 1) * L] = jnp.zeros((L,), jnp.float32)
        return carry
    lax.fori_loop(0, B, _zrow, 0)
    row0 = sid * RPT
    for j in range(RPT // B):
        pltpu.make_async_copy(row_v, acc_sh.at[pl.ds(row0 + j * B, B)],
                              s_sc0).start()
    for j in range(RPT // B):
        pltpu.make_async_copy(row_v, acc_sh.at[pl.ds(row0, B)],
                              s_sc0).wait()

    def _zs(i, carry):
        s0_v[pl.ds(i * L, L)] = jnp.zeros((L,), jnp.float32)
        return carry
    lax.fori_loop(0, NP // L, _zs, 0)
    plsc.subcore_barrier()

    pltpu.sync_copy(att_hbm, att_v)
    att_c = [att_v[cc * L:(cc + 1) * L] for cc in range(C // L)]
    lane = lax.iota(jnp.int32, L)
    lane0 = lane == 0
    bfly = [jnp.bitwise_xor(lane, sh).reshape(L, 1) for sh in (8, 4, 2, 1)]
    gd = lax.GatherDimensionNumbers(
        offset_dims=(), collapsed_slice_dims=(0,), start_index_map=(0,))

    def _xl_sum(v):
        for idx in bfly:
            v = v + lax.gather(v, idx, gd, slice_sizes=(1,),
                               mode=lax.GatherScatterMode.PROMISE_IN_BOUNDS)
        return v

    ebase = (cid * NS + sid) * EPT

    def _issue_idx(k, b):
        eoff = pl.multiple_of(ebase + k * B, 8)
        pltpu.make_async_copy(src_hbm.at[pl.ds(eoff, B)], b[0], b[4]).start()
        pltpu.make_async_copy(dst_hbm.at[pl.ds(eoff, B)], b[1], b[5]).start()
        pltpu.make_async_copy(ea_hbm.at[pl.ds(eoff, B)], b[3], b[8]).start()

    def _wait_idx(b):
        pltpu.make_async_copy(src_hbm.at[pl.ds(0, B)], b[0], b[4]).wait()
        pltpu.make_async_copy(dst_hbm.at[pl.ds(0, B)], b[1], b[5]).wait()
        pltpu.make_async_copy(ea_hbm.at[pl.ds(0, B)], b[3], b[8]).wait()

    def _issue_data(k, b):
        pltpu.make_async_copy(xl_hbm.at[b[0]], b[2], b[6]).start()
        pltpu.make_async_copy(xr_hbm.at[b[1]], b[3], b[7]).start(add=True)

    def _wait_data(b):
        pltpu.make_async_copy(xl_hbm.at[b[0]], b[2], b[6]).wait()
        pltpu.make_async_copy(xr_hbm.at[b[1]], b[3], b[7]).wait()

    def _compute_scatter(b, first):
        dst_b, xj_b, ea_b = b[1], b[2], b[3]
        rv, sidx, ssc = b[9], b[10], b[11]

        @pl.when(jnp.logical_not(first))
        def _():
            pltpu.make_async_copy(rv, acc_sh.at[sidx], ssc).wait()

        def _edge(e, ecarry):
            acc = jnp.zeros((L,), jnp.float32)
            xjs = []
            for cc in range(C // L):
                vj = xj_b[e, cc * L:(cc + 1) * L]
                ve = ea_b[e, cc * L:(cc + 1) * L]
                v = vj + ve
                v = jnp.where(v > 0, v, SLOPE * v)
                acc = acc + v * att_c[cc]
                xjs.append(vj)
            w = jnp.exp(_xl_sum(acc))
            for cc in range(C // L):
                rv[e, cc * L:(cc + 1) * L] = xjs[cc] * w
            plsc.store_scatter(w_v, [jnp.full((L,), e, jnp.int32)], w,
                               mask=lane0)
            return ecarry
        lax.fori_loop(0, B, _edge, 0)

        for g in range((B + L - 1) // L):
            off = min(g * L, B - L)
            wv = w_v[off:off + L]
            dv = dst_b[off:off + L]
            if off == g * L:
                plsc.addupdate_scatter(s0_v, [dv], wv)
            else:
                plsc.addupdate_scatter(s0_v, [dv], wv,
                                       mask=lane >= (g * L - off))

        for g in (0, L, B - L):
            sidx[g:g + L] = dst_b[g:g + L]
        pltpu.make_async_copy(rv, acc_sh.at[sidx], ssc).start(add=True)

    _issue_idx(0, bufs[0])
    _issue_idx(1, bufs[1])
    _wait_idx(bufs[0])
    _issue_data(0, bufs[0])

    def _body(k2, carry):
        for slot in (0, 1):
            k = k2 * 2 + slot
            b = bufs[slot]
            o = bufs[1 - slot]

            @pl.when(k < NCHUNK - 1)
            def _():
                _wait_idx(o)
                _issue_data(k + 1, o)

            _wait_data(b)
            _compute_scatter(b, k2 == 0)

            @pl.when(k < NCHUNK - 2)
            def _():
                _issue_idx(k + 2, b)
        return carry
    lax.fori_loop(0, NCHUNK // 2, _body, 0)

    pltpu.make_async_copy(row_v0, acc_sh.at[sidx0], s_sc0).wait()
    pltpu.make_async_copy(row_v1, acc_sh.at[sidx1], s_sc1).wait()

    pltpu.sync_copy(s0_v, s0_hbm.at[cid, sid])

    plsc.subcore_barrier()
    pltpu.sync_copy(acc_sh.at[pl.ds(row0, RPT)],
                    out_hbm.at[cid, pl.ds(row0, RPT)])


@functools.partial(
    pl.kernel,
    mesh=plsc.VectorSubcoreMesh(core_axis_name="c", subcore_axis_name="s"),
    compiler_params=pltpu.CompilerParams(needs_layout_passes=False),
    out_type=[
        jax.ShapeDtypeStruct((NC, NP, R), jnp.float32),
        jax.ShapeDtypeStruct((NC, NS, NP), jnp.float32),
    ],
    scratch_types=[
        pltpu.VMEM((B,), jnp.int32),
        pltpu.VMEM((B,), jnp.int32),
        pltpu.VMEM((B,), jnp.int32),
        pltpu.VMEM((B,), jnp.int32),
        pltpu.VMEM((B, C), jnp.float32),
        pltpu.VMEM((B, C), jnp.float32),
        pltpu.VMEM((B, C), jnp.float32),
        pltpu.VMEM((B, C), jnp.float32),
        pltpu.VMEM((B, C), jnp.float32),
        pltpu.VMEM((B, C), jnp.float32),
        pltpu.VMEM((B,), jnp.int32),
        pltpu.VMEM((B,), jnp.int32),
        pltpu.VMEM((B,), jnp.float32),
        pltpu.VMEM((C,), jnp.float32),
        pltpu.VMEM((NP,), jnp.float32),
        pltpu.VMEM_SHARED((NP, R), jnp.float32),
    ] + [pltpu.SemaphoreType.DMA] * 12,
)
def _edge_pass(src_hbm, dst_hbm, xl_hbm, xr_hbm, ea_hbm, att_hbm, out_hbm,
               s0_hbm,
               src_v0, dst_v0, src_v1, dst_v1,
               xj_v0, ea_v0, xj_v1, ea_v1,
               row_v0, row_v1, sidx0, sidx1, w_v, att_v, s0_v, acc_sh,
               s_si0, s_di0, s_gj0, s_gi0, s_ge0,
               s_si1, s_di1, s_gj1, s_gi1, s_ge1, s_sc0, s_sc1):
    _edge_pass_body(src_hbm, dst_hbm, xl_hbm, xr_hbm, ea_hbm, att_hbm,
                    out_hbm, s0_hbm,
                    src_v0, dst_v0, src_v1, dst_v1,
                    xj_v0, ea_v0, xj_v1, ea_v1,
                    row_v0, row_v1, sidx0, sidx1, w_v, att_v, s0_v, acc_sh,
                    s_si0, s_di0, s_gj0, s_gi0, s_ge0,
                    s_si1, s_di1, s_gj1, s_gi1, s_ge1, s_sc0, s_sc1)


def kernel(x, edge_index, edge_attr, Wl1, bl1, Wr1, br1, We1, att1, bo1,
           Wl2, bl2, Wr2, br2, We2, att2, bo2, Wd1, bd1, Wd2, bd2):
    src = edge_index[0]
    dst = edge_index[1]

    xl1, xr1 = _node_xfm(x, Wl1, bl1, Wr1, br1)
    eaw1 = _eaw(edge_attr, We1)
    acc1, s01 = _edge_pass(src, dst, xl1, xr1, eaw1, att1.reshape(C))

    eaw2 = _eaw(edge_attr, We2)
    xl2, xr2 = _head(acc1, s01, bo1, Wl2, bl2, Wr2, br2)
    acc2, s02 = _edge_pass(src, dst, xl2, xr2, eaw2, att2.reshape(C))

    return _final(acc2, s02, bo2, Wd1, bd1, Wd2, bd2)[:N]


# speedup vs baseline: 1.0815x; 1.0008x over previous
import functools

import jax
import jax.numpy as jnp
from jax import lax
from jax.experimental import pallas as pl
from jax.experimental.pallas import tpu as pltpu
from jax.experimental.pallas import tpu_sc as plsc

N = 10000
E = 320000
C = 128
ED = 16
DD = 128
DO = 2
SLOPE = 0.2
NL = 0.01

NC = 2
NS = 16
L = 16

R = C
NP = 10240
EPT = E // (NC * NS)
B = 40
RPT = NP // NS


def _node_xfm_body(x_ref, wl_ref, bl_ref, wr_ref, br_ref, xl_ref, xr_ref):
    xb = x_ref[...]
    xl_ref[...] = jnp.dot(xb, wl_ref[...], preferred_element_type=jnp.float32) + bl_ref[...]
    xr_ref[...] = jnp.dot(xb, wr_ref[...], preferred_element_type=jnp.float32) + br_ref[...]


def _node_xfm(x, Wl, bl, Wr, br):
    bn = 1000
    grid = (N // bn,)
    return pl.pallas_call(
        _node_xfm_body,
        grid=grid,
        in_specs=[
            pl.BlockSpec((bn, C), lambda i: (i, 0)),
            pl.BlockSpec((C, C), lambda i: (0, 0)),
            pl.BlockSpec((1, C), lambda i: (0, 0)),
            pl.BlockSpec((C, C), lambda i: (0, 0)),
            pl.BlockSpec((1, C), lambda i: (0, 0)),
        ],
        out_specs=[
            pl.BlockSpec((bn, C), lambda i: (i, 0)),
            pl.BlockSpec((bn, C), lambda i: (i, 0)),
        ],
        out_shape=[
            jax.ShapeDtypeStruct((N, C), jnp.float32),
            jax.ShapeDtypeStruct((N, C), jnp.float32),
        ],
    )(x, Wl, bl.reshape(1, C), Wr, br.reshape(1, C))


def _eaw_body(ea_ref, we_ref, out_ref):
    out_ref[...] = jnp.dot(ea_ref[...], we_ref[...], preferred_element_type=jnp.float32)


def _eaw(edge_attr, We):
    be = 4000
    grid = (E // be,)
    return pl.pallas_call(
        _eaw_body,
        grid=grid,
        in_specs=[
            pl.BlockSpec((be, ED), lambda i: (i, 0)),
            pl.BlockSpec((ED, C), lambda i: (0, 0)),
        ],
        out_specs=pl.BlockSpec((be, C), lambda i: (i, 0)),
        out_shape=jax.ShapeDtypeStruct((E, C), jnp.float32),
    )(edge_attr, We)


def _norm_h(acc_ref, s0_ref, bo_ref):
    acc_blk = acc_ref[...]
    num = acc_blk[0] + acc_blk[1]
    s = jnp.sum(s0_ref[...], axis=(0, 1))
    h = num / (s + 1e-16) + bo_ref[...]
    return jnp.where(h > 0, h, NL * h)


def _head_body(acc_ref, s0_ref, bo_ref, wl_ref, bl_ref, wr_ref, br_ref, xl_ref, xr_ref):
    h = _norm_h(acc_ref, s0_ref, bo_ref)
    xl_ref[...] = jnp.dot(h, wl_ref[...], preferred_element_type=jnp.float32) + bl_ref[...]
    xr_ref[...] = jnp.dot(h, wr_ref[...], preferred_element_type=jnp.float32) + br_ref[...]


def _head(acc, s0, bo, Wl, bl, Wr, br):
    bn = 1024
    grid = (NP // bn,)
    return pl.pallas_call(
        _head_body,
        grid=grid,
        in_specs=[
            pl.BlockSpec((NC, bn, C), lambda i: (0, i, 0)),
            pl.BlockSpec((NC, NS, bn, 1), lambda i: (0, 0, i, 0)),
            pl.BlockSpec((1, C), lambda i: (0, 0)),
            pl.BlockSpec((C, C), lambda i: (0, 0)),
            pl.BlockSpec((1, C), lambda i: (0, 0)),
            pl.BlockSpec((C, C), lambda i: (0, 0)),
            pl.BlockSpec((1, C), lambda i: (0, 0)),
        ],
        out_specs=[
            pl.BlockSpec((bn, C), lambda i: (i, 0)),
            pl.BlockSpec((bn, C), lambda i: (i, 0)),
        ],
        out_shape=[
            jax.ShapeDtypeStruct((NP, C), jnp.float32),
            jax.ShapeDtypeStruct((NP, C), jnp.float32),
        ],
    )(acc, s0.reshape(NC, NS, NP, 1), bo.reshape(1, C), Wl, bl.reshape(1, C),
      Wr, br.reshape(1, C))


def _final_body(acc_ref, s0_ref, bo_ref, wd1_ref, bd1_ref, wd2_ref, bd2_ref,
                out_ref):
    h = _norm_h(acc_ref, s0_ref, bo_ref)
    d = jnp.dot(h, wd1_ref[...], preferred_element_type=jnp.float32) + bd1_ref[...]
    d = jnp.where(d > 0, d, NL * d)
    out_ref[...] = jnp.dot(d, wd2_ref[...], preferred_element_type=jnp.float32) + bd2_ref[...]


def _final(acc, s0, bo, Wd1, bd1, Wd2, bd2):
    bn = 1024
    grid = (NP // bn,)
    return pl.pallas_call(
        _final_body,
        grid=grid,
        in_specs=[
            pl.BlockSpec((NC, bn, C), lambda i: (0, i, 0)),
            pl.BlockSpec((NC, NS, bn, 1), lambda i: (0, 0, i, 0)),
            pl.BlockSpec((1, C), lambda i: (0, 0)),
            pl.BlockSpec((C, DD), lambda i: (0, 0)),
            pl.BlockSpec((1, DD), lambda i: (0, 0)),
            pl.BlockSpec((DD, DO), lambda i: (0, 0)),
            pl.BlockSpec((1, DO), lambda i: (0, 0)),
        ],
        out_specs=pl.BlockSpec((bn, DO), lambda i: (i, 0)),
        out_shape=jax.ShapeDtypeStruct((NP, DO), jnp.float32),
    )(acc, s0.reshape(NC, NS, NP, 1), bo.reshape(1, C), Wd1,
      bd1.reshape(1, DD), Wd2, bd2.reshape(1, DO))


NCHUNK = EPT // B


def _edge_pass_body(src_hbm, dst_hbm, xl_hbm, xr_hbm, ea_hbm, att_hbm,
                    out_hbm, s0_hbm,
                    src_v0, dst_v0, src_v1, dst_v1,
                    xj_v0, ea_v0, xj_v1, ea_v1,
                    row_v0, row_v1, sidx0, sidx1, w_v, att_v, s0_v, acc_sh,
                    s_si0, s_di0, s_gj0, s_gi0, s_ge0,
                    s_si1, s_di1, s_gj1, s_gi1, s_ge1, s_sc0, s_sc1):
    cid = lax.axis_index("c")
    row_v = row_v0
    sid = lax.axis_index("s")

    bufs = (
        (src_v0, dst_v0, xj_v0, ea_v0, s_si0, s_di0, s_gj0, s_gi0, s_ge0,
         row_v0, sidx0, s_sc0),
        (src_v1, dst_v1, xj_v1, ea_v1, s_si1, s_di1, s_gj1, s_gi1, s_ge1,
         row_v1, sidx1, s_sc1),
    )

    def _zrow(i, carry):
        for cc in range(R // L):
            row_v[i, cc * L:(cc + 1) * L] = jnp.zeros((L,), jnp.float32)
        return carry
    lax.fori_loop(0, B, _zrow, 0)
    row0 = sid * RPT
    for j in range(RPT // B):
        pltpu.make_async_copy(row_v, acc_sh.at[pl.ds(row0 + j * B, B)],
                              s_sc0).start()
    for j in range(RPT // B):
        pltpu.make_async_copy(row_v, acc_sh.at[pl.ds(row0, B)],
                              s_sc0).wait()

    def _zs(i, carry):
        s0_v[pl.ds(i * L, L)] = jnp.zeros((L,), jnp.float32)
        return carry
    lax.fori_loop(0, NP // L, _zs, 0)
    plsc.subcore_barrier()

    pltpu.sync_copy(att_hbm, att_v)
    att_c = [att_v[cc * L:(cc + 1) * L] for cc in range(C // L)]
    lane = lax.iota(jnp.int32, L)
    lane0 = lane == 0
    bfly = [jnp.bitwise_xor(lane, sh).reshape(L, 1) for sh in (8, 4, 2, 1)]
    gd = lax.GatherDimensionNumbers(
        offset_dims=(), collapsed_slice_dims=(0,), start_index_map=(0,))

    def _xl_sum(v):
        for idx in bfly:
            v = v + lax.gather(v, idx, gd, slice_sizes=(1,),
                               mode=lax.GatherScatterMode.PROMISE_IN_BOUNDS)
        return v

    ebase = (cid * NS + sid) * EPT

    def _issue_idx(k, b):
        eoff = pl.multiple_of(ebase + k * B, 8)
        pltpu.make_async_copy(src_hbm.at[pl.ds(eoff, B)], b[0], b[4]).start()
        pltpu.make_async_copy(dst_hbm.at[pl.ds(eoff, B)], b[1], b[5]).start()
        pltpu.make_async_copy(ea_hbm.at[pl.ds(eoff, B)], b[3], b[8]).start()

    def _wait_idx(b):
        pltpu.make_async_copy(src_hbm.at[pl.ds(0, B)], b[0], b[4]).wait()
        pltpu.make_async_copy(dst_hbm.at[pl.ds(0, B)], b[1], b[5]).wait()
        pltpu.make_async_copy(ea_hbm.at[pl.ds(0, B)], b[3], b[8]).wait()

    def _issue_data(k, b):
        pltpu.make_async_copy(xl_hbm.at[b[0]], b[2], b[6]).start()
        pltpu.make_async_copy(xr_hbm.at[b[1]], b[3], b[7]).start(add=True)

    def _wait_data(b):
        pltpu.make_async_copy(xl_hbm.at[b[0]], b[2], b[6]).wait()
        pltpu.make_async_copy(xr_hbm.at[b[1]], b[3], b[7]).wait()

    def _compute_scatter(b, first):
        dst_b, xj_b, ea_b = b[1], b[2], b[3]
        rv, sidx, ssc = b[9], b[10], b[11]

        @pl.when(jnp.logical_not(first))
        def _():
            pltpu.make_async_copy(rv, acc_sh.at[sidx], ssc).wait()

        def _edge(e, ecarry):
            acc = jnp.zeros((L,), jnp.float32)
            xjs = []
            for cc in range(C // L):
                vj = xj_b[e, cc * L:(cc + 1) * L]
                ve = ea_b[e, cc * L:(cc + 1) * L]
                v = vj + ve
                v = jnp.where(v > 0, v, SLOPE * v)
                acc = acc + v * att_c[cc]
                xjs.append(vj)
            w = jnp.exp(_xl_sum(acc))
            for cc in range(C // L):
                rv[e, cc * L:(cc + 1) * L] = xjs[cc] * w
            plsc.store_scatter(w_v, [jnp.full((L,), e, jnp.int32)], w,
                               mask=lane0)
            return ecarry
        lax.fori_loop(0, B, _edge, 0)

        for g in (0, L, B - L):
            sidx[g:g + L] = dst_b[g:g + L]
        pltpu.make_async_copy(rv, acc_sh.at[sidx], ssc).start(add=True)

        for g in range((B + L - 1) // L):
            off = min(g * L, B - L)
            wv = w_v[off:off + L]
            dv = dst_b[off:off + L]
            if off == g * L:
                plsc.addupdate_scatter(s0_v, [dv], wv)
            else:
                plsc.addupdate_scatter(s0_v, [dv], wv,
                                       mask=lane >= (g * L - off))

    _issue_idx(0, bufs[0])
    _issue_idx(1, bufs[1])
    _wait_idx(bufs[0])
    _issue_data(0, bufs[0])

    def _body(k2, carry):
        for slot in (0, 1):
            k = k2 * 2 + slot
            b = bufs[slot]
            o = bufs[1 - slot]

            @pl.when(k < NCHUNK - 1)
            def _():
                _wait_idx(o)
                _issue_data(k + 1, o)

            _wait_data(b)
            _compute_scatter(b, k2 == 0)

            @pl.when(k < NCHUNK - 2)
            def _():
                _issue_idx(k + 2, b)
        return carry
    lax.fori_loop(0, NCHUNK // 2, _body, 0)

    pltpu.make_async_copy(row_v0, acc_sh.at[sidx0], s_sc0).wait()
    pltpu.make_async_copy(row_v1, acc_sh.at[sidx1], s_sc1).wait()

    pltpu.sync_copy(s0_v, s0_hbm.at[cid, sid])

    plsc.subcore_barrier()
    pltpu.sync_copy(acc_sh.at[pl.ds(row0, RPT)],
                    out_hbm.at[cid, pl.ds(row0, RPT)])


@functools.partial(
    pl.kernel,
    mesh=plsc.VectorSubcoreMesh(core_axis_name="c", subcore_axis_name="s"),
    compiler_params=pltpu.CompilerParams(needs_layout_passes=False),
    out_type=[
        jax.ShapeDtypeStruct((NC, NP, R), jnp.float32),
        jax.ShapeDtypeStruct((NC, NS, NP), jnp.float32),
    ],
    scratch_types=[
        pltpu.VMEM((B,), jnp.int32),
        pltpu.VMEM((B,), jnp.int32),
        pltpu.VMEM((B,), jnp.int32),
        pltpu.VMEM((B,), jnp.int32),
        pltpu.VMEM((B, C), jnp.float32),
        pltpu.VMEM((B, C), jnp.float32),
        pltpu.VMEM((B, C), jnp.float32),
        pltpu.VMEM((B, C), jnp.float32),
        pltpu.VMEM((B, C), jnp.float32),
        pltpu.VMEM((B, C), jnp.float32),
        pltpu.VMEM((B,), jnp.int32),
        pltpu.VMEM((B,), jnp.int32),
        pltpu.VMEM((B,), jnp.float32),
        pltpu.VMEM((C,), jnp.float32),
        pltpu.VMEM((NP,), jnp.float32),
        pltpu.VMEM_SHARED((NP, R), jnp.float32),
    ] + [pltpu.SemaphoreType.DMA] * 12,
)
def _edge_pass(src_hbm, dst_hbm, xl_hbm, xr_hbm, ea_hbm, att_hbm, out_hbm,
               s0_hbm,
               src_v0, dst_v0, src_v1, dst_v1,
               xj_v0, ea_v0, xj_v1, ea_v1,
               row_v0, row_v1, sidx0, sidx1, w_v, att_v, s0_v, acc_sh,
               s_si0, s_di0, s_gj0, s_gi0, s_ge0,
               s_si1, s_di1, s_gj1, s_gi1, s_ge1, s_sc0, s_sc1):
    _edge_pass_body(src_hbm, dst_hbm, xl_hbm, xr_hbm, ea_hbm, att_hbm,
                    out_hbm, s0_hbm,
                    src_v0, dst_v0, src_v1, dst_v1,
                    xj_v0, ea_v0, xj_v1, ea_v1,
                    row_v0, row_v1, sidx0, sidx1, w_v, att_v, s0_v, acc_sh,
                    s_si0, s_di0, s_gj0, s_gi0, s_ge0,
                    s_si1, s_di1, s_gj1, s_gi1, s_ge1, s_sc0, s_sc1)


def kernel(x, edge_index, edge_attr, Wl1, bl1, Wr1, br1, We1, att1, bo1,
           Wl2, bl2, Wr2, br2, We2, att2, bo2, Wd1, bd1, Wd2, bd2):
    src = edge_index[0]
    dst = edge_index[1]

    xl1, xr1 = _node_xfm(x, Wl1, bl1, Wr1, br1)
    eaw1 = _eaw(edge_attr, We1)
    acc1, s01 = _edge_pass(src, dst, xl1, xr1, eaw1, att1.reshape(C))

    eaw2 = _eaw(edge_attr, We2)
    xl2, xr2 = _head(acc1, s01, bo1, Wl2, bl2, Wr2, br2)
    acc2, s02 = _edge_pass(src, dst, xl2, xr2, eaw2, att2.reshape(C))

    return _final(acc2, s02, bo2, Wd1, bd1, Wd2, bd2)[:N]
